# baseline (speedup 1.0000x reference)
import functools

import jax
import jax.numpy as jnp
from jax import lax
from jax.experimental import pallas as pl
from jax.experimental.pallas import tpu as pltpu
from jax.experimental.pallas import tpu_sc as plsc

_NC = 2
_NS = 16
_NW = _NC * _NS
_L = 16
_SLOTW = 128


@functools.lru_cache(maxsize=None)
def _make_sc(B, D, NNEG, interpret=False):
    RPS = _SLOTW // D
    CHUNK = B // _NW
    S = 16
    NSUB = CHUNK // S
    SN = S * NNEG
    KD = D // _L
    NC_CH = CHUNK * NNEG
    mesh = plsc.VectorSubcoreMesh(core_axis_name="c", subcore_axis_name="s",
                                  num_cores=_NC, num_subcores=_NS)

    @functools.partial(
        pl.kernel, mesh=mesh, interpret=interpret,
        out_type=(jax.ShapeDtypeStruct((B // 8, 8 * _L), jnp.float32),
                  jax.ShapeDtypeStruct((B // 8, 8 * _L), jnp.float32)),
        scratch_types=[
            pltpu.VMEM((CHUNK,), jnp.int32),
            pltpu.VMEM((CHUNK,), jnp.int32),
            pltpu.VMEM((NC_CH,), jnp.int32),
            pltpu.VMEM((CHUNK,), jnp.int32),
            pltpu.VMEM((CHUNK,), jnp.int32),
            pltpu.VMEM((NC_CH,), jnp.int32),
            pltpu.VMEM((S, _SLOTW), jnp.float32),
            pltpu.VMEM((S, _SLOTW), jnp.float32),
            pltpu.VMEM((SN, _SLOTW), jnp.float32),
            pltpu.VMEM((CHUNK // 8, 8 * _L), jnp.float32),
            pltpu.VMEM((CHUNK // 8, 8 * _L), jnp.float32),
            pltpu.SemaphoreType.DMA,
        ],
    )
    def sc_fn(upos, vpos, vnegf, uw2, vw2, spart, npart,
              uidx, vidx, nidx, usix, vsix, nsix,
              uslot, vslot, nslot, sbuf, nbuf, sem):
        wid = lax.axis_index("s") * _NC + lax.axis_index("c")
        base = wid * CHUNK

        pltpu.sync_copy(upos.at[pl.ds(base, CHUNK)], uidx)
        pltpu.sync_copy(vpos.at[pl.ds(base, CHUNK)], vidx)
        pltpu.sync_copy(vnegf.at[pl.ds(base * NNEG, NC_CH)], nidx)

        def shift_body(i, _):
            o = pl.multiple_of(i * _L, _L)
            usix[pl.ds(o, _L)] = lax.shift_right_logical(uidx[pl.ds(o, _L)], 1)
            vsix[pl.ds(o, _L)] = lax.shift_right_logical(vidx[pl.ds(o, _L)], 1)
            return _
        lax.fori_loop(0, CHUNK // _L, shift_body, 0)

        def nshift_body(i, _):
            o = pl.multiple_of(i * _L, _L)
            nsix[pl.ds(o, _L)] = lax.shift_right_logical(nidx[pl.ds(o, _L)], 1)
            return _
        lax.fori_loop(0, NC_CH // _L, nshift_body, 0)

        def sub(j, carry):
            jS = pl.multiple_of(j * S, S)
            jSN = pl.multiple_of(j * SN, SN)
            cps = [pltpu.async_copy(uw2.at[usix.at[pl.ds(jS, S)]], uslot, sem),
                   pltpu.async_copy(vw2.at[vsix.at[pl.ds(jS, S)]], vslot, sem)]
            done = 0
            while done < SN:
                c = min(128, SN - done)
                cps.append(pltpu.async_copy(
                    vw2.at[nsix.at[pl.ds(jSN + done, c)]],
                    nslot.at[pl.ds(done, c)], sem))
                done += c
            for cp in cps:
                cp.wait()

            ruv = (uidx[pl.ds(jS, _L)] & (RPS - 1)) * D
            rvv = (vidx[pl.ds(jS, _L)] & (RPS - 1)) * D
            rnv = [(nidx[pl.ds(jSN + c * _L, _L)] & (RPS - 1)) * D
                   for c in range(SN // _L)]

            for b in range(S):
                ru = ruv[b]
                rv = rvv[b]
                su = [uslot[b, pl.ds(ru + k * _L, _L)] for k in range(KD)]
                sv = [vslot[b, pl.ds(rv + k * _L, _L)] for k in range(KD)]
                ps = su[0] * sv[0]
                for k in range(1, KD):
                    ps = ps + su[k] * sv[k]
                row = 2 * j + b // 8
                sbuf[row, pl.ds((b % 8) * _L, _L)] = ps
                acc = None
                for n in range(NNEG):
                    q = b * NNEG + n
                    rn = rnv[q // _L][q % _L]
                    for k in range(KD):
                        t = nslot[q, pl.ds(rn + k * _L, _L)] * su[k]
                        acc = t if acc is None else acc + t
                nbuf[row, pl.ds((b % 8) * _L, _L)] = acc
            return carry

        lax.fori_loop(0, NSUB, sub, 0)
        rbase = wid * (CHUNK // 8)
        pltpu.sync_copy(sbuf, spart.at[pl.ds(rbase, CHUNK // 8)])
        pltpu.sync_copy(nbuf, npart.at[pl.ds(rbase, CHUNK // 8)])

    return sc_fn


def _tc_finish(spart, npart, group, interpret=False):
    def body(s_ref, n_ref, g_ref, o_ref):
        g = g_ref[...]
        s = jax.lax.dot(s_ref[...], g, preferred_element_type=jnp.float32)
        ns = jax.lax.dot(n_ref[...], g, preferred_element_type=jnp.float32)
        tot = jnp.sum(jax.nn.log_sigmoid(s) + jax.nn.log_sigmoid(-ns))
        o_ref[0, 0] = tot
    out = pl.pallas_call(
        body,
        out_shape=jax.ShapeDtypeStruct((1, 1), jnp.float32),
        out_specs=pl.BlockSpec(memory_space=pltpu.SMEM),
        interpret=interpret,
    )(spart, npart, group)
    return out[0, 0]


def kernel(u_pos, v_pos, v_neg, batch_size, u_weight, v_weight):
    B = u_pos.shape[0]
    D = u_weight.shape[1]
    NNEG = v_neg.shape[1]
    up = u_pos.astype(jnp.int32)
    vp = v_pos.astype(jnp.int32)
    vn = v_neg.astype(jnp.int32).reshape(-1)
    uw2 = u_weight.reshape(-1, _SLOTW)
    vw2 = v_weight.reshape(-1, _SLOTW)
    spart, npart = _make_sc(B, D, NNEG)(up, vp, vn, uw2, vw2)
    group = (jnp.arange(8 * _L)[:, None] // _L
             == jnp.arange(8)[None, :]).astype(jnp.float32)
    tot = _tc_finish(spart, npart, group)
    return -tot / batch_size


# speedup vs baseline: 2.0240x; 2.0240x over previous
import functools

import jax
import jax.numpy as jnp
from jax import lax
from jax.experimental import pallas as pl
from jax.experimental.pallas import tpu as pltpu
from jax.experimental.pallas import tpu_sc as plsc

_NC = 2
_NS = 16
_NW = _NC * _NS
_L = 16
_SLOTW = 128


@functools.lru_cache(maxsize=None)
def _make_sc(B, D, NNEG, interpret=False):
    RPS = _SLOTW // D
    CHUNK = B // _NW
    S = 16
    NSUB = CHUNK // S
    SN = S * NNEG
    KD = D // _L
    NC_CH = CHUNK * NNEG
    mesh = plsc.VectorSubcoreMesh(core_axis_name="c", subcore_axis_name="s",
                                  num_cores=_NC, num_subcores=_NS)

    @functools.partial(
        pl.kernel, mesh=mesh, interpret=interpret,
        out_type=(jax.ShapeDtypeStruct((B // 8, 8 * _L), jnp.float32),
                  jax.ShapeDtypeStruct((B // 8, 8 * _L), jnp.float32)),
        scratch_types=[
            pltpu.VMEM((CHUNK,), jnp.int32),
            pltpu.VMEM((CHUNK,), jnp.int32),
            pltpu.VMEM((NC_CH,), jnp.int32),
            pltpu.VMEM((CHUNK,), jnp.int32),
            pltpu.VMEM((CHUNK,), jnp.int32),
            pltpu.VMEM((NC_CH,), jnp.int32),
            pltpu.VMEM((S, _SLOTW), jnp.float32),
            pltpu.VMEM((S, _SLOTW), jnp.float32),
            pltpu.VMEM((SN, _SLOTW), jnp.float32),
            pltpu.VMEM((CHUNK // 8, 8 * _L), jnp.float32),
            pltpu.VMEM((CHUNK // 8, 8 * _L), jnp.float32),
            pltpu.SemaphoreType.DMA,
        ],
    )
    def sc_fn(upos, vpos, vnegf, uw2, vw2, spart, npart,
              uidx, vidx, nidx, usix, vsix, nsix,
              uslot, vslot, nslot, sbuf, nbuf, sem):
        wid = lax.axis_index("s") * _NC + lax.axis_index("c")
        base = wid * CHUNK

        pltpu.sync_copy(upos.at[pl.ds(base, CHUNK)], uidx)
        pltpu.sync_copy(vpos.at[pl.ds(base, CHUNK)], vidx)
        pltpu.sync_copy(vnegf.at[pl.ds(base * NNEG, NC_CH)], nidx)

        def to_slot(a):
            hi = lax.shift_left(lax.shift_right_logical(a, _CB + 1), _CB)
            return hi | (a & (_CBLK - 1))

        def shift_body(i, _):
            o = pl.multiple_of(i * _L, _L)
            usix[pl.ds(o, _L)] = to_slot(uidx[pl.ds(o, _L)])
            vsix[pl.ds(o, _L)] = to_slot(vidx[pl.ds(o, _L)])
            return _
        lax.fori_loop(0, CHUNK // _L, shift_body, 0)

        def nshift_body(i, _):
            o = pl.multiple_of(i * _L, _L)
            nsix[pl.ds(o, _L)] = to_slot(nidx[pl.ds(o, _L)])
            return _
        lax.fori_loop(0, NC_CH // _L, nshift_body, 0)

        def sub(j, carry):
            jS = pl.multiple_of(j * S, S)
            jSN = pl.multiple_of(j * SN, SN)
            cps = [pltpu.async_copy(uw2.at[usix.at[pl.ds(jS, S)]], uslot, sem),
                   pltpu.async_copy(vw2.at[vsix.at[pl.ds(jS, S)]], vslot, sem)]
            done = 0
            while done < SN:
                c = min(128, SN - done)
                cps.append(pltpu.async_copy(
                    vw2.at[nsix.at[pl.ds(jSN + done, c)]],
                    nslot.at[pl.ds(done, c)], sem))
                done += c
            for cp in cps:
                cp.wait()

            def to_half(a):
                return (lax.shift_right_logical(a, _CB) & 1) * D

            ruv = to_half(uidx[pl.ds(jS, _L)])
            rvv = to_half(vidx[pl.ds(jS, _L)])
            rnv = [to_half(nidx[pl.ds(jSN + c * _L, _L)])
                   for c in range(SN // _L)]

            for b in range(S):
                ru = ruv[b]
                rv = rvv[b]
                su = [uslot[b, pl.ds(ru + k * _L, _L)] for k in range(KD)]
                sv = [vslot[b, pl.ds(rv + k * _L, _L)] for k in range(KD)]
                ps = su[0] * sv[0]
                for k in range(1, KD):
                    ps = ps + su[k] * sv[k]
                row = 2 * j + b // 8
                sbuf[row, pl.ds((b % 8) * _L, _L)] = ps
                acc = None
                for n in range(NNEG):
                    q = b * NNEG + n
                    rn = rnv[q // _L][q % _L]
                    for k in range(KD):
                        t = nslot[q, pl.ds(rn + k * _L, _L)] * su[k]
                        acc = t if acc is None else acc + t
                nbuf[row, pl.ds((b % 8) * _L, _L)] = acc
            return carry

        lax.fori_loop(0, NSUB, sub, 0)
        rbase = wid * (CHUNK // 8)
        pltpu.sync_copy(sbuf, spart.at[pl.ds(rbase, CHUNK // 8)])
        pltpu.sync_copy(nbuf, npart.at[pl.ds(rbase, CHUNK // 8)])

    return sc_fn


_CB = 13
_CBLK = 1 << _CB


@functools.lru_cache(maxsize=None)
def _make_relayout(NF, V, interpret=False):
    nblk = (V + 2 * _CBLK - 1) // (2 * _CBLK)
    SLOTS = nblk * _CBLK
    last = (V - 1) // _CBLK

    def body(x1_ref, x2_ref, o_ref):
        o_ref[:, 0:NF] = x1_ref[...].T
        o_ref[:, NF:2 * NF] = x2_ref[...].T

    f = pl.pallas_call(
        body,
        grid=(nblk,),
        in_specs=[pl.BlockSpec((NF, _CBLK),
                               lambda i: (0, jnp.minimum(2 * i, last))),
                  pl.BlockSpec((NF, _CBLK),
                               lambda i: (0, jnp.minimum(2 * i + 1, last)))],
        out_specs=pl.BlockSpec((_CBLK, 2 * NF), lambda i: (i, 0)),
        out_shape=jax.ShapeDtypeStruct((SLOTS, 2 * NF), jnp.float32),
        interpret=interpret,
    )
    return lambda xT: f(xT, xT)


def _tc_finish(spart, npart, group, interpret=False):
    def body(s_ref, n_ref, g_ref, o_ref):
        g = g_ref[...]
        s = jax.lax.dot(s_ref[...], g, preferred_element_type=jnp.float32)
        ns = jax.lax.dot(n_ref[...], g, preferred_element_type=jnp.float32)
        tot = jnp.sum(jax.nn.log_sigmoid(s) + jax.nn.log_sigmoid(-ns))
        o_ref[0, 0] = tot
    out = pl.pallas_call(
        body,
        out_shape=jax.ShapeDtypeStruct((1, 1), jnp.float32),
        out_specs=pl.BlockSpec(memory_space=pltpu.SMEM),
        interpret=interpret,
    )(spart, npart, group)
    return out[0, 0]


def kernel(u_pos, v_pos, v_neg, batch_size, u_weight, v_weight):
    B = u_pos.shape[0]
    D = u_weight.shape[1]
    NNEG = v_neg.shape[1]
    up = u_pos.astype(jnp.int32)
    vp = v_pos.astype(jnp.int32)
    vn = v_neg.astype(jnp.int32).reshape(-1)
    V = u_weight.shape[0]
    relay = _make_relayout(D, V)
    uw2 = relay(u_weight.T)
    vw2 = relay(v_weight.T)
    spart, npart = _make_sc(B, D, NNEG)(up, vp, vn, uw2, vw2)
    group = (jnp.arange(8 * _L)[:, None] // _L
             == jnp.arange(8)[None, :]).astype(jnp.float32)
    tot = _tc_finish(spart, npart, group)
    return -tot / batch_size


# speedup vs baseline: 2.3673x; 1.1696x over previous
import functools

import jax
import jax.numpy as jnp
from jax import lax
from jax.experimental import pallas as pl
from jax.experimental.pallas import tpu as pltpu
from jax.experimental.pallas import tpu_sc as plsc

_NC = 2
_NS = 16
_NW = _NC * _NS
_L = 16
_SLOTW = 128


def _to_slot(a):
    hi = lax.shift_left(lax.shift_right_logical(a, _CB + 1), _CB)
    return hi | (a & (_CBLK - 1))


def _to_half(a, D):
    return (lax.shift_right_logical(a, _CB) & 1) * D


def _mesh():
    return plsc.VectorSubcoreMesh(core_axis_name="c", subcore_axis_name="s",
                                  num_cores=_NC, num_subcores=_NS)


@functools.lru_cache(maxsize=None)
def _make_sc_vneg(B, D, NNEG, interpret=False):
    CHUNK = B // _NW
    S = 16
    NSUB = CHUNK // S
    SN = S * NNEG
    KD = D // _L
    NC_CH = CHUNK * NNEG

    @functools.partial(
        pl.kernel, mesh=_mesh(), interpret=interpret,
        out_type=jax.ShapeDtypeStruct((B, 2 * D), jnp.float32),
        scratch_types=[
            pltpu.VMEM((CHUNK,), jnp.int32),
            pltpu.VMEM((NC_CH,), jnp.int32),
            pltpu.VMEM((CHUNK,), jnp.int32),
            pltpu.VMEM((NC_CH,), jnp.int32),
            pltpu.VMEM((S, _SLOTW), jnp.float32),
            pltpu.VMEM((SN, _SLOTW), jnp.float32),
            pltpu.VMEM((S, 2 * D), jnp.float32),
            pltpu.SemaphoreType.DMA,
        ],
    )
    def sc_fn(vpos, vnegf, vw2, pack,
              vidx, nidx, vsix, nsix, vslot, nslot, pbuf, sem):
        wid = lax.axis_index("s") * _NC + lax.axis_index("c")
        base = wid * CHUNK

        pltpu.sync_copy(vpos.at[pl.ds(base, CHUNK)], vidx)
        pltpu.sync_copy(vnegf.at[pl.ds(base * NNEG, NC_CH)], nidx)

        def shift_body(i, _):
            o = pl.multiple_of(i * _L, _L)
            vsix[pl.ds(o, _L)] = _to_slot(vidx[pl.ds(o, _L)])
            return _
        lax.fori_loop(0, CHUNK // _L, shift_body, 0)

        def nshift_body(i, _):
            o = pl.multiple_of(i * _L, _L)
            nsix[pl.ds(o, _L)] = _to_slot(nidx[pl.ds(o, _L)])
            return _
        lax.fori_loop(0, NC_CH // _L, nshift_body, 0)

        def sub(j, carry):
            jS = pl.multiple_of(j * S, S)
            jSN = pl.multiple_of(j * SN, SN)
            cps = [pltpu.async_copy(vw2.at[vsix.at[pl.ds(jS, S)]], vslot, sem)]
            done = 0
            while done < SN:
                c = min(128, SN - done)
                cps.append(pltpu.async_copy(
                    vw2.at[nsix.at[pl.ds(jSN + done, c)]],
                    nslot.at[pl.ds(done, c)], sem))
                done += c
            for cp in cps:
                cp.wait()

            rvv = _to_half(vidx[pl.ds(jS, _L)], D)
            rnv = [_to_half(nidx[pl.ds(jSN + c * _L, _L)], D)
                   for c in range(SN // _L)]

            for b in range(S):
                rv = rvv[b]
                for k in range(KD):
                    pbuf[b, pl.ds(k * _L, _L)] = vslot[b, pl.ds(rv + k * _L, _L)]
                for k in range(KD):
                    acc = None
                    for n in range(NNEG):
                        q = b * NNEG + n
                        rn = rnv[q // _L][q % _L]
                        t = nslot[q, pl.ds(rn + k * _L, _L)]
                        acc = t if acc is None else acc + t
                    pbuf[b, pl.ds(D + k * _L, _L)] = acc
            pltpu.sync_copy(pbuf, pack.at[pl.ds(base + jS, S)])
            return carry

        lax.fori_loop(0, NSUB, sub, 0)

    return sc_fn


@functools.lru_cache(maxsize=None)
def _make_sc_dots(B, D, interpret=False):
    CHUNK = B // _NW
    S = 16
    NSUB = CHUNK // S
    KD = D // _L

    @functools.partial(
        pl.kernel, mesh=_mesh(), interpret=interpret,
        out_type=(jax.ShapeDtypeStruct((B // 8, 8 * _L), jnp.float32),
                  jax.ShapeDtypeStruct((B // 8, 8 * _L), jnp.float32)),
        scratch_types=[
            pltpu.VMEM((CHUNK,), jnp.int32),
            pltpu.VMEM((CHUNK,), jnp.int32),
            pltpu.VMEM((S, _SLOTW), jnp.float32),
            pltpu.VMEM((S, 2 * D), jnp.float32),
            pltpu.VMEM((CHUNK // 8, 8 * _L), jnp.float32),
            pltpu.VMEM((CHUNK // 8, 8 * _L), jnp.float32),
            pltpu.SemaphoreType.DMA,
        ],
    )
    def sc_fn(upos, pack, uw2, spart, npart,
              uidx, usix, uslot, pkslot, sbuf, nbuf, sem):
        wid = lax.axis_index("s") * _NC + lax.axis_index("c")
        base = wid * CHUNK

        pltpu.sync_copy(upos.at[pl.ds(base, CHUNK)], uidx)

        def shift_body(i, _):
            o = pl.multiple_of(i * _L, _L)
            usix[pl.ds(o, _L)] = _to_slot(uidx[pl.ds(o, _L)])
            return _
        lax.fori_loop(0, CHUNK // _L, shift_body, 0)

        def sub(j, carry):
            jS = pl.multiple_of(j * S, S)
            cps = [pltpu.async_copy(uw2.at[usix.at[pl.ds(jS, S)]], uslot, sem)]
            pltpu.sync_copy(pack.at[pl.ds(base + jS, S)], pkslot)
            for cp in cps:
                cp.wait()

            ruv = _to_half(uidx[pl.ds(jS, _L)], D)

            for b in range(S):
                ru = ruv[b]
                su = [uslot[b, pl.ds(ru + k * _L, _L)] for k in range(KD)]
                ps = None
                pn = None
                for k in range(KD):
                    tv = pkslot[b, pl.ds(k * _L, _L)] * su[k]
                    tn = pkslot[b, pl.ds(D + k * _L, _L)] * su[k]
                    ps = tv if ps is None else ps + tv
                    pn = tn if pn is None else pn + tn
                row = 2 * j + b // 8
                sbuf[row, pl.ds((b % 8) * _L, _L)] = ps
                nbuf[row, pl.ds((b % 8) * _L, _L)] = pn
            return carry

        lax.fori_loop(0, NSUB, sub, 0)
        rbase = wid * (CHUNK // 8)
        pltpu.sync_copy(sbuf, spart.at[pl.ds(rbase, CHUNK // 8)])
        pltpu.sync_copy(nbuf, npart.at[pl.ds(rbase, CHUNK // 8)])

    return sc_fn


_CB = 13
_CBLK = 1 << _CB


@functools.lru_cache(maxsize=None)
def _make_relayout(NF, V, interpret=False):
    nblk = (V + 2 * _CBLK - 1) // (2 * _CBLK)
    SLOTS = nblk * _CBLK
    last = (V - 1) // _CBLK

    def body(x1_ref, x2_ref, o_ref):
        o_ref[:, 0:NF] = x1_ref[...].T
        o_ref[:, NF:2 * NF] = x2_ref[...].T

    f = pl.pallas_call(
        body,
        grid=(nblk,),
        in_specs=[pl.BlockSpec((NF, _CBLK),
                               lambda i: (0, jnp.minimum(2 * i, last))),
                  pl.BlockSpec((NF, _CBLK),
                               lambda i: (0, jnp.minimum(2 * i + 1, last)))],
        out_specs=pl.BlockSpec((_CBLK, 2 * NF), lambda i: (i, 0)),
        out_shape=jax.ShapeDtypeStruct((SLOTS, 2 * NF), jnp.float32),
        interpret=interpret,
    )
    return lambda xT: f(xT, xT)


def _tc_finish(spart, npart, group, interpret=False):
    def body(s_ref, n_ref, g_ref, o_ref):
        g = g_ref[...]
        s = jax.lax.dot(s_ref[...], g, preferred_element_type=jnp.float32)
        ns = jax.lax.dot(n_ref[...], g, preferred_element_type=jnp.float32)
        tot = jnp.sum(jax.nn.log_sigmoid(s) + jax.nn.log_sigmoid(-ns))
        o_ref[0, 0] = tot
    out = pl.pallas_call(
        body,
        out_shape=jax.ShapeDtypeStruct((1, 1), jnp.float32),
        out_specs=pl.BlockSpec(memory_space=pltpu.SMEM),
        interpret=interpret,
    )(spart, npart, group)
    return out[0, 0]


def kernel(u_pos, v_pos, v_neg, batch_size, u_weight, v_weight):
    B = u_pos.shape[0]
    D = u_weight.shape[1]
    NNEG = v_neg.shape[1]
    up = u_pos.astype(jnp.int32)
    vp = v_pos.astype(jnp.int32)
    vn = v_neg.astype(jnp.int32).reshape(-1)
    V = u_weight.shape[0]
    relay = _make_relayout(D, V)
    vw2 = relay(v_weight.T)
    pack = _make_sc_vneg(B, D, NNEG)(vp, vn, vw2)
    uw2 = relay(u_weight.T)
    spart, npart = _make_sc_dots(B, D)(up, pack, uw2)
    group = (jnp.arange(8 * _L)[:, None] // _L
             == jnp.arange(8)[None, :]).astype(jnp.float32)
    tot = _tc_finish(spart, npart, group)
    return -tot / batch_size


# speedup vs baseline: 2.8514x; 1.2045x over previous
import functools

import jax
import jax.numpy as jnp
from jax import lax
from jax.experimental import pallas as pl
from jax.experimental.pallas import tpu as pltpu
from jax.experimental.pallas import tpu_sc as plsc

_NC = 2
_NS = 16
_NW = _NC * _NS
_L = 16
_SLOTW = 128


def _to_slot(a):
    hi = lax.shift_left(lax.shift_right_logical(a, _CB + 1), _CB)
    return hi | (a & (_CBLK - 1))


def _to_half(a, D):
    return (lax.shift_right_logical(a, _CB) & 1) * D


def _mesh():
    return plsc.VectorSubcoreMesh(core_axis_name="c", subcore_axis_name="s",
                                  num_cores=_NC, num_subcores=_NS)


@functools.lru_cache(maxsize=None)
def _make_sc_vneg(B, D, NNEG, interpret=False):
    CHUNK = B // _NW
    S = 16
    NSUB = CHUNK // S
    SN = S * NNEG
    KD = D // _L
    NC_CH = CHUNK * NNEG

    @functools.partial(
        pl.kernel, mesh=_mesh(), interpret=interpret,
        out_type=jax.ShapeDtypeStruct((B, 2 * D), jnp.float32),
        scratch_types=[
            pltpu.VMEM((CHUNK,), jnp.int32),
            pltpu.VMEM((NC_CH,), jnp.int32),
            pltpu.VMEM((CHUNK,), jnp.int32),
            pltpu.VMEM((NC_CH,), jnp.int32),
            pltpu.VMEM((S, _SLOTW), jnp.float32),
            pltpu.VMEM((SN, _SLOTW), jnp.float32),
            pltpu.VMEM((S, 2 * D), jnp.float32),
            pltpu.SemaphoreType.DMA,
        ],
    )
    def sc_fn(vpos, vnegf, vw2, pack,
              vidx, nidx, vsix, nsix, vslot, nslot, pbuf, sem):
        wid = lax.axis_index("s") * _NC + lax.axis_index("c")
        base = wid * CHUNK

        pltpu.sync_copy(vpos.at[pl.ds(base, CHUNK)], vidx)
        pltpu.sync_copy(vnegf.at[pl.ds(base * NNEG, NC_CH)], nidx)

        def shift_body(i, _):
            o = pl.multiple_of(i * _L, _L)
            vsix[pl.ds(o, _L)] = _to_slot(vidx[pl.ds(o, _L)])
            return _
        lax.fori_loop(0, CHUNK // _L, shift_body, 0)

        def nshift_body(i, _):
            o = pl.multiple_of(i * _L, _L)
            nsix[pl.ds(o, _L)] = _to_slot(nidx[pl.ds(o, _L)])
            return _
        lax.fori_loop(0, NC_CH // _L, nshift_body, 0)

        def sub(j, carry):
            jS = pl.multiple_of(j * S, S)
            jSN = pl.multiple_of(j * SN, SN)
            cps = [pltpu.async_copy(vw2.at[vsix.at[pl.ds(jS, S)]], vslot, sem)]
            done = 0
            while done < SN:
                c = min(128, SN - done)
                cps.append(pltpu.async_copy(
                    vw2.at[nsix.at[pl.ds(jSN + done, c)]],
                    nslot.at[pl.ds(done, c)], sem))
                done += c
            for cp in cps:
                cp.wait()

            rvv = _to_half(vidx[pl.ds(jS, _L)], D)
            rnv = [_to_half(nidx[pl.ds(jSN + c * _L, _L)], D)
                   for c in range(SN // _L)]

            for b in range(S):
                rv = rvv[b]
                for k in range(KD):
                    pbuf[b, pl.ds(k * _L, _L)] = vslot[b, pl.ds(rv + k * _L, _L)]
                for k in range(KD):
                    acc = None
                    for n in range(NNEG):
                        q = b * NNEG + n
                        rn = rnv[q // _L][q % _L]
                        t = nslot[q, pl.ds(rn + k * _L, _L)]
                        acc = t if acc is None else acc + t
                    pbuf[b, pl.ds(D + k * _L, _L)] = acc
            pltpu.sync_copy(pbuf, pack.at[pl.ds(base + jS, S)])
            return carry

        lax.fori_loop(0, NSUB, sub, 0)

    return sc_fn


@functools.lru_cache(maxsize=None)
def _make_sc_dots(B, D, interpret=False):
    CHUNK = B // _NW
    S = 16
    NSUB = CHUNK // S
    KD = D // _L

    @functools.partial(
        pl.kernel, mesh=_mesh(), interpret=interpret,
        out_type=(jax.ShapeDtypeStruct((B // 8, 8 * _L), jnp.float32),
                  jax.ShapeDtypeStruct((B // 8, 8 * _L), jnp.float32)),
        scratch_types=[
            pltpu.VMEM((CHUNK,), jnp.int32),
            pltpu.VMEM((CHUNK,), jnp.int32),
            pltpu.VMEM((S, _SLOTW), jnp.float32),
            pltpu.VMEM((S, 2 * D), jnp.float32),
            pltpu.VMEM((CHUNK // 8, 8 * _L), jnp.float32),
            pltpu.VMEM((CHUNK // 8, 8 * _L), jnp.float32),
            pltpu.SemaphoreType.DMA,
        ],
    )
    def sc_fn(upos, pack, uw2, spart, npart,
              uidx, usix, uslot, pkslot, sbuf, nbuf, sem):
        wid = lax.axis_index("s") * _NC + lax.axis_index("c")
        base = wid * CHUNK

        pltpu.sync_copy(upos.at[pl.ds(base, CHUNK)], uidx)

        def shift_body(i, _):
            o = pl.multiple_of(i * _L, _L)
            usix[pl.ds(o, _L)] = _to_slot(uidx[pl.ds(o, _L)])
            return _
        lax.fori_loop(0, CHUNK // _L, shift_body, 0)

        def sub(j, carry):
            jS = pl.multiple_of(j * S, S)
            cps = [pltpu.async_copy(uw2.at[usix.at[pl.ds(jS, S)]], uslot, sem)]
            pltpu.sync_copy(pack.at[pl.ds(base + jS, S)], pkslot)
            for cp in cps:
                cp.wait()

            ruv = _to_half(uidx[pl.ds(jS, _L)], D)

            for b in range(S):
                ru = ruv[b]
                su = [uslot[b, pl.ds(ru + k * _L, _L)] for k in range(KD)]
                ps = None
                pn = None
                for k in range(KD):
                    tv = pkslot[b, pl.ds(k * _L, _L)] * su[k]
                    tn = pkslot[b, pl.ds(D + k * _L, _L)] * su[k]
                    ps = tv if ps is None else ps + tv
                    pn = tn if pn is None else pn + tn
                row = 2 * j + b // 8
                sbuf[row, pl.ds((b % 8) * _L, _L)] = ps
                nbuf[row, pl.ds((b % 8) * _L, _L)] = pn
            return carry

        lax.fori_loop(0, NSUB, sub, 0)
        rbase = wid * (CHUNK // 8)
        pltpu.sync_copy(sbuf, spart.at[pl.ds(rbase, CHUNK // 8)])
        pltpu.sync_copy(nbuf, npart.at[pl.ds(rbase, CHUNK // 8)])

    return sc_fn


_CB = 13
_CBLK = 1 << _CB


@functools.lru_cache(maxsize=None)
def _make_relayout(NF, V, interpret=False):
    nblk = (V + 2 * _CBLK - 1) // (2 * _CBLK)
    SLOTS = nblk * _CBLK
    last = (V - 1) // _CBLK

    def body(x1_ref, x2_ref, i_ref, o_ref):
        x = jnp.concatenate([x1_ref[...], x2_ref[...]], axis=0)
        o_ref[...] = lax.dot_general(
            x, i_ref[...], (((0,), (0,)), ((), ())),
            preferred_element_type=jnp.float32)

    f = pl.pallas_call(
        body,
        grid=(nblk,),
        in_specs=[pl.BlockSpec((NF, _CBLK),
                               lambda i: (0, jnp.minimum(2 * i, last))),
                  pl.BlockSpec((NF, _CBLK),
                               lambda i: (0, jnp.minimum(2 * i + 1, last))),
                  pl.BlockSpec((2 * NF, 2 * NF), lambda i: (0, 0))],
        out_specs=pl.BlockSpec((_CBLK, 2 * NF), lambda i: (i, 0)),
        out_shape=jax.ShapeDtypeStruct((SLOTS, 2 * NF), jnp.float32),
        compiler_params=pltpu.CompilerParams(
            fuse_transposed_lhs_in_matmul=True),
        interpret=interpret,
    )
    ident = jnp.eye(2 * NF, dtype=jnp.float32)
    return lambda xT: f(xT, xT, ident)


def _tc_finish(spart, npart, group, interpret=False):
    def body(s_ref, n_ref, g_ref, o_ref):
        g = g_ref[...]
        s = jax.lax.dot(s_ref[...], g, preferred_element_type=jnp.float32)
        ns = jax.lax.dot(n_ref[...], g, preferred_element_type=jnp.float32)
        tot = jnp.sum(jax.nn.log_sigmoid(s) + jax.nn.log_sigmoid(-ns))
        o_ref[0, 0] = tot
    out = pl.pallas_call(
        body,
        out_shape=jax.ShapeDtypeStruct((1, 1), jnp.float32),
        out_specs=pl.BlockSpec(memory_space=pltpu.SMEM),
        interpret=interpret,
    )(spart, npart, group)
    return out[0, 0]


def kernel(u_pos, v_pos, v_neg, batch_size, u_weight, v_weight):
    B = u_pos.shape[0]
    D = u_weight.shape[1]
    NNEG = v_neg.shape[1]
    up = u_pos.astype(jnp.int32)
    vp = v_pos.astype(jnp.int32)
    vn = v_neg.astype(jnp.int32).reshape(-1)
    V = u_weight.shape[0]
    relay = _make_relayout(D, V)
    vw2 = relay(v_weight.T)
    pack = _make_sc_vneg(B, D, NNEG)(vp, vn, vw2)
    uw2 = relay(u_weight.T)
    spart, npart = _make_sc_dots(B, D)(up, pack, uw2)
    group = (jnp.arange(8 * _L)[:, None] // _L
             == jnp.arange(8)[None, :]).astype(jnp.float32)
    tot = _tc_finish(spart, npart, group)
    return -tot / batch_size
